# baseline (speedup 1.0000x reference)
import dataclasses
import functools

import jax
import jax.numpy as jnp
from jax import lax
from jax.experimental import pallas as pl
from jax.experimental.pallas import tpu as pltpu
from jax.experimental.pallas import tpu_sc as plsc

B = 4096
F = 26
V = 100000
E = 16
D = F * E
BF = B * F

NC = 2
NS = 16
NW = NC * NS
ROWS_PER_W = BF // NW

RPC = 4
FPC = RPC * F
CHUNKS = (B // NW) // RPC


SR = V // 8
SRP = 12504
PCH = 48
NPCH = 261
PLAST = SR - 260 * PCH
PLASTW = SRP - 260 * PCH


def _sc_pack(table_deep, table_lr):
    mesh = plsc.VectorSubcoreMesh(core_axis_name="c", subcore_axis_name="s")
    cp = pltpu.CompilerParams()
    if "needs_layout_passes" in pltpu.CompilerParams.__dataclass_fields__:
        cp = dataclasses.replace(cp, needs_layout_passes=False)

    @functools.partial(
        pl.kernel,
        mesh=mesh,
        compiler_params=cp,
        out_type=jax.ShapeDtypeStruct((SRP, 256), jnp.float32),
        scratch_types=[
            pltpu.VMEM((PCH * 8, E), jnp.float32),
            pltpu.VMEM((PCH, 128), jnp.float32),
        ],
    )
    def k(deep_hbm, lr_hbm, packed_hbm, inb, outb):
        cid = lax.axis_index("c")
        sid = lax.axis_index("s")

        def do_chunk(c, nsr, nwr):
            base = c * PCH

            @pl.when(cid == 0)
            def _():
                pltpu.sync_copy(deep_hbm.at[pl.ds(base * 8, nsr * 8)],
                                inb.at[pl.ds(0, nsr * 8)])

            @pl.when(cid == 1)
            def _():
                pltpu.sync_copy(lr_hbm.at[pl.ds(base * 8, nsr * 8)],
                                inb.at[pl.ds(0, nsr * 8)])

            @pl.loop(0, nsr)
            def _row(rl):
                for kk in range(8):
                    outb[rl, pl.ds(kk * 16, 16)] = inb[rl * 8 + kk, :]

            pltpu.sync_copy(
                outb.at[pl.ds(0, nwr)],
                packed_hbm.at[pl.ds(base, nwr), pl.ds(cid * 128, 128)])

        @pl.loop(0, (NPCH + NS - 1) // NS)
        def _outer(kk):
            c = kk * NS + sid

            @pl.when(c < NPCH - 1)
            def _():
                do_chunk(c, PCH, PCH)

            @pl.when(c == NPCH - 1)
            def _():
                do_chunk(c, PLAST, PLASTW)

    return k(table_deep, table_lr)


def _sc_gather2(packed, idx_flat):
    mesh = plsc.VectorSubcoreMesh(core_axis_name="c", subcore_axis_name="s")
    cp = pltpu.CompilerParams()
    if "needs_layout_passes" in pltpu.CompilerParams.__dataclass_fields__:
        cp = dataclasses.replace(cp, needs_layout_passes=False)

    @functools.partial(
        pl.kernel,
        mesh=mesh,
        compiler_params=cp,
        out_type=(jax.ShapeDtypeStruct((B, D), jnp.float32),
                  jax.ShapeDtypeStruct((B, D), jnp.float32)),
        scratch_types=[
            pltpu.VMEM((ROWS_PER_W,), jnp.int32),
            pltpu.VMEM((ROWS_PER_W,), jnp.int32),
            pltpu.VMEM((ROWS_PER_W,), jnp.int32),
            pltpu.VMEM((2, FPC, 256), jnp.float32),
            pltpu.VMEM((2, RPC, D), jnp.float32),
            pltpu.VMEM((2, RPC, D), jnp.float32),
            pltpu.SemaphoreType.DMA,
            pltpu.SemaphoreType.DMA,
            pltpu.SemaphoreType.DMA,
            pltpu.SemaphoreType.DMA,
        ],
    )
    def k(tab_hbm, idx_hbm, outd_hbm, outw_hbm, idx_v, sidx_v, off_v,
          rows_v, outd_s, outw_s, gsem0, gsem1, osem0, osem1):
        wid = lax.axis_index("s") * NC + lax.axis_index("c")
        flat_base = wid * ROWS_PER_W
        obase = wid * (B // NW)
        iota16 = jax.lax.iota(jnp.int32, 16)
        gsems = (gsem0, gsem1)
        osems = (osem0, osem1)

        pltpu.sync_copy(idx_hbm.at[pl.ds(flat_base, ROWS_PER_W)], idx_v)
        for r16 in range(ROWS_PER_W // 16):
            s = slice(r16 * 16, r16 * 16 + 16)
            v = idx_v[s]
            sidx_v[s] = jax.lax.shift_right_logical(v, 3)
            off_v[s] = jax.lax.shift_left(jax.lax.bitwise_and(v, 7), 4)

        def issue_gather(ci, buf):
            pltpu.async_copy(
                tab_hbm.at[sidx_v.at[pl.ds(ci * FPC, FPC)]],
                rows_v.at[buf], gsems[buf])

        def wait_gather(buf):
            pltpu.make_async_copy(tab_hbm.at[pl.ds(0, FPC)],
                                  rows_v.at[buf], gsems[buf]).wait()

        def wait_out(ci, buf):
            pltpu.make_async_copy(
                outd_s.at[buf],
                outd_hbm.at[pl.ds(obase + ci * RPC, RPC)], osems[buf]).wait()
            pltpu.make_async_copy(
                outw_s.at[buf],
                outw_hbm.at[pl.ds(obase + ci * RPC, RPC)], osems[buf]).wait()

        def select_and_store(ci, buf):
            rows_b = rows_v.at[buf]
            outd_b = outd_s.at[buf]
            outw_b = outw_s.at[buf]

            @pl.loop(0, RPC)
            def _row(rl):
                coff = ci * FPC
                for f in range(F):
                    fr = rl * F + f
                    fr_vec = jnp.full((16,), fr, jnp.int32)
                    off_b = plsc.load_gather(off_v, [fr_vec + coff])
                    col = off_b + iota16
                    outd_b[rl, pl.ds(f * 16, 16)] = plsc.load_gather(
                        rows_b, [fr_vec, col])
                    outw_b[rl, pl.ds(f * 16, 16)] = plsc.load_gather(
                        rows_b, [fr_vec, col + 128])

            pltpu.async_copy(
                outd_b, outd_hbm.at[pl.ds(obase + ci * RPC, RPC)],
                osems[buf])
            pltpu.async_copy(
                outw_b, outw_hbm.at[pl.ds(obase + ci * RPC, RPC)],
                osems[buf])

        issue_gather(0, 0)
        issue_gather(1, 1)

        @pl.loop(0, CHUNKS, step=2)
        def _chunk(ci):
            for b in range(2):
                cib = ci + b

                @pl.when(cib >= 2)
                def _():
                    wait_out(cib - 2, b)

                wait_gather(b)
                select_and_store(cib, b)

                @pl.when(cib + 2 < CHUNKS)
                def _():
                    issue_gather(cib + 2, b)

        wait_out(CHUNKS - 2, 0)
        wait_out(CHUNKS - 1, 1)

    return k(packed, idx_flat)


def _mlp_body(deep_ref, g_ref, be_ref, w1_ref, b1_ref, w2_ref, b2_ref,
              w3_ref, b3_ref, d_ref):
    x = deep_ref[...]
    mean = jnp.mean(x, axis=0, keepdims=True)
    cent = x - mean
    var = jnp.mean(cent * cent, axis=0, keepdims=True)
    xn = cent * lax.rsqrt(var + 1e-5) * g_ref[...] + be_ref[...]
    bf = jnp.bfloat16
    h = jnp.dot(xn.astype(bf), w1_ref[...].astype(bf),
                preferred_element_type=jnp.float32)
    h = jnp.maximum(h + b1_ref[...], 0.0)
    h = jnp.dot(h.astype(bf), w2_ref[...].astype(bf),
                preferred_element_type=jnp.float32)
    h = jnp.maximum(h + b2_ref[...], 0.0)
    d_ref[...] = (jnp.sum(h * w3_ref[...], axis=1, keepdims=True)
                  + b3_ref[...])


def _mlp(deep, gamma, beta, W1, b1, W2, b2, w3row, b3):
    return pl.pallas_call(
        _mlp_body,
        out_shape=jax.ShapeDtypeStruct((B, 1), jnp.float32),
    )(deep, gamma, beta, W1, b1, W2, b2, w3row, b3)


def _combine_body(w_ref, d_ref, o_ref):
    o_ref[...] = jax.nn.sigmoid(w_ref[...] + d_ref[...])


def _combine(wide, d):
    return pl.pallas_call(
        _combine_body,
        out_shape=jax.ShapeDtypeStruct((B, D), jnp.float32),
    )(wide, d)


def kernel(x, table_lr, table_deep, gamma, beta, W1, b1, W2, b2, W3, b3):
    idx_flat = x.reshape(BF)
    packed = _sc_pack(table_deep, table_lr)
    deep, wide = _sc_gather2(packed, idx_flat)
    d = _mlp(deep,
             gamma.reshape(1, D), beta.reshape(1, D),
             W1, b1.reshape(1, 1024), W2, b2.reshape(1, 512),
             W3.reshape(1, 512), b3.reshape(1, 1))
    return _combine(wide, d)


# speedup vs baseline: 1.4039x; 1.4039x over previous
import dataclasses
import functools

import jax
import jax.numpy as jnp
from jax import lax
from jax.experimental import pallas as pl
from jax.experimental.pallas import tpu as pltpu
from jax.experimental.pallas import tpu_sc as plsc

B = 4096
F = 26
V = 100000
E = 16
D = F * E
BF = B * F

NC = 2
NS = 16
NW = NC * NS
ROWS_PER_W = BF // NW


ROWS_PER_CHUNK = 8
FLAT_PER_CHUNK = ROWS_PER_CHUNK * F
CHUNKS_PER_W = (B // NW) // ROWS_PER_CHUNK


def _sc_gather(table128, idx_flat):
    mesh = plsc.VectorSubcoreMesh(core_axis_name="c", subcore_axis_name="s")
    cp = pltpu.CompilerParams()
    if "needs_layout_passes" in pltpu.CompilerParams.__dataclass_fields__:
        cp = dataclasses.replace(cp, needs_layout_passes=False)

    @functools.partial(
        pl.kernel,
        mesh=mesh,
        compiler_params=cp,
        out_type=jax.ShapeDtypeStruct((B, D), jnp.float32),
        scratch_types=[
            pltpu.VMEM((ROWS_PER_W,), jnp.int32),
            pltpu.VMEM((ROWS_PER_W,), jnp.int32),
            pltpu.VMEM((ROWS_PER_W,), jnp.int32),
            pltpu.VMEM((2, FLAT_PER_CHUNK, 128), jnp.float32),
            pltpu.VMEM((2, ROWS_PER_CHUNK, D), jnp.float32),
            pltpu.SemaphoreType.DMA,
            pltpu.SemaphoreType.DMA,
            pltpu.SemaphoreType.DMA,
            pltpu.SemaphoreType.DMA,
        ],
    )
    def k(table_hbm, idx_hbm, out_hbm, idx_v, sidx_v, off_v, rows_v,
          out_s, gsem0, gsem1, osem0, osem1):
        wid = lax.axis_index("s") * NC + lax.axis_index("c")
        flat_base = wid * ROWS_PER_W
        obase = wid * (B // NW)
        iota16 = jax.lax.iota(jnp.int32, 16)
        gsems = (gsem0, gsem1)
        osems = (osem0, osem1)

        pltpu.sync_copy(idx_hbm.at[pl.ds(flat_base, ROWS_PER_W)], idx_v)
        for r16 in range(ROWS_PER_W // 16):
            s = slice(r16 * 16, r16 * 16 + 16)
            v = idx_v[s]
            sidx_v[s] = jax.lax.shift_right_logical(v, 3)
            off_v[s] = jax.lax.shift_left(jax.lax.bitwise_and(v, 7), 4)

        def issue_gather(ci, buf):
            pltpu.async_copy(
                table_hbm.at[sidx_v.at[pl.ds(ci * FLAT_PER_CHUNK,
                                             FLAT_PER_CHUNK)]],
                rows_v.at[buf], gsems[buf])

        def wait_gather(buf):
            pltpu.make_async_copy(table_hbm.at[pl.ds(0, FLAT_PER_CHUNK)],
                                  rows_v.at[buf], gsems[buf]).wait()

        def wait_out(ci, buf):
            pltpu.make_async_copy(
                out_s.at[buf],
                out_hbm.at[pl.ds(obase + ci * ROWS_PER_CHUNK,
                                 ROWS_PER_CHUNK)], osems[buf]).wait()

        def select_and_store(ci, buf):
            rows_b = rows_v.at[buf]
            out_b = out_s.at[buf]

            @pl.loop(0, ROWS_PER_CHUNK)
            def _row(rl):
                coff = ci * FLAT_PER_CHUNK
                for f in range(F):
                    fr = rl * F + f
                    fr_vec = jnp.full((16,), fr, jnp.int32)
                    off_b = plsc.load_gather(off_v, [fr_vec + coff])
                    out_b[rl, pl.ds(f * 16, 16)] = plsc.load_gather(
                        rows_b, [fr_vec, off_b + iota16])

            pltpu.async_copy(
                out_b, out_hbm.at[pl.ds(obase + ci * ROWS_PER_CHUNK,
                                        ROWS_PER_CHUNK)], osems[buf])

        issue_gather(0, 0)
        issue_gather(1, 1)

        @pl.loop(0, CHUNKS_PER_W, step=2)
        def _chunk(ci):
            for b in range(2):
                cib = ci + b

                @pl.when(cib >= 2)
                def _():
                    wait_out(cib - 2, b)

                wait_gather(b)
                select_and_store(cib, b)

                @pl.when(cib + 2 < CHUNKS_PER_W)
                def _():
                    issue_gather(cib + 2, b)

        wait_out(CHUNKS_PER_W - 2, 0)
        wait_out(CHUNKS_PER_W - 1, 1)

    return k(table128, idx_flat)


def _mlp_body(deep_ref, g_ref, be_ref, w1_ref, b1_ref, w2_ref, b2_ref,
              w3_ref, b3_ref, d_ref):
    x = deep_ref[...]
    mean = jnp.mean(x, axis=0, keepdims=True)
    cent = x - mean
    var = jnp.mean(cent * cent, axis=0, keepdims=True)
    xn = cent * lax.rsqrt(var + 1e-5) * g_ref[...] + be_ref[...]
    bf = jnp.bfloat16
    h = jnp.dot(xn.astype(bf), w1_ref[...].astype(bf),
                preferred_element_type=jnp.float32)
    h = jnp.maximum(h + b1_ref[...], 0.0)
    h = jnp.dot(h.astype(bf), w2_ref[...].astype(bf),
                preferred_element_type=jnp.float32)
    h = jnp.maximum(h + b2_ref[...], 0.0)
    d_ref[...] = (jnp.sum(h * w3_ref[...], axis=1, keepdims=True)
                  + b3_ref[...])


def _mlp(deep, gamma, beta, W1, b1, W2, b2, w3row, b3):
    return pl.pallas_call(
        _mlp_body,
        out_shape=jax.ShapeDtypeStruct((B, 1), jnp.float32),
    )(deep, gamma, beta, W1, b1, W2, b2, w3row, b3)


def _combine_body(w_ref, d_ref, o_ref):
    o_ref[...] = jax.nn.sigmoid(w_ref[...] + d_ref[...])


def _combine(wide, d):
    return pl.pallas_call(
        _combine_body,
        out_shape=jax.ShapeDtypeStruct((B, D), jnp.float32),
    )(wide, d)


def kernel(x, table_lr, table_deep, gamma, beta, W1, b1, W2, b2, W3, b3):
    idx_flat = x.reshape(BF)
    deep = _sc_gather(table_deep.reshape(V // 8, 128), idx_flat)
    wide = _sc_gather(table_lr.reshape(V // 8, 128), idx_flat)
    d = _mlp(deep,
             gamma.reshape(1, D), beta.reshape(1, D),
             W1, b1.reshape(1, 1024), W2, b2.reshape(1, 512),
             W3.reshape(1, 512), b3.reshape(1, 1))
    return _combine(wide, d)
